# Initial kernel scaffold; baseline (speedup 1.0000x reference)
import jax
import jax.numpy as jnp
from jax.experimental import pallas as pl


def kernel(R_ij, i, j, Z_i, node_to_graph, pair_mask, node_mask, embed_table, W1, b1, W2, b2, W_tr, b_tr, ln1_s, ln1_b, Wq1, bq1, Wq2, bq2, W3, b3, ln2_s, ln2_b):
    raise NotImplementedError("write your pallas kernel here")


# speedup vs baseline: 5.9861x; 5.9861x over previous
import functools
import math

import jax
import jax.numpy as jnp
from jax import lax
from jax.experimental import pallas as pl
from jax.experimental.pallas import tpu as pltpu
from jax.experimental.pallas import tpu_sc as plsc

N = 10000
E = 160000
NR = 32
NSP = 4
NSC = 16
NLM = 9
CUT = 5.0

NWORK = 32
EW = E // NWORK
CH = 40
KCH = EW // CH
NSTRIPE = N // 10

EBLK = 2000
EGRID = E // EBLK

_LOG_BINOM = [
    math.lgamma(NR) - math.lgamma(v + 1.0) - math.lgamma(NR - v)
    for v in range(NR)
]


def _sc_gather2_body(tab, i2h, j2h, oi, oj, iv, jv, rv):
  c = lax.axis_index("c")
  s = lax.axis_index("s")
  w = s * 2 + c
  pltpu.sync_copy(i2h.at[w], iv)
  pltpu.sync_copy(j2h.at[w], jv)

  def step(k, carry):
    base = w * EW + k * CH
    pltpu.sync_copy(tab.at[iv.at[k]], rv)
    pltpu.sync_copy(rv, oi.at[pl.ds(base, CH)])
    pltpu.sync_copy(tab.at[jv.at[k]], rv)
    pltpu.sync_copy(rv, oj.at[pl.ds(base, CH)])
    return carry

  lax.fori_loop(0, KCH, step, 0)


def _sc_gather1_body(tab, i2h, oi, iv, rv):
  c = lax.axis_index("c")
  s = lax.axis_index("s")
  w = s * 2 + c
  pltpu.sync_copy(i2h.at[w], iv)

  def step(k, carry):
    pltpu.sync_copy(tab.at[iv.at[k]], rv)
    pltpu.sync_copy(rv, oi.at[pl.ds(w * EW + k * CH, CH)])
    return carry

  lax.fori_loop(0, KCH, step, 0)


def _sc_scatter_body(rows, i2h, zrows, out, iv, rv, acc):
  c = lax.axis_index("c")
  s = lax.axis_index("s")
  w = s * 2 + c
  pltpu.sync_copy(i2h.at[w], iv)

  @pl.when(s < 10)
  def _():
    pltpu.sync_copy(zrows, acc.at[pl.ds(s * NSTRIPE, NSTRIPE)])

  plsc.subcore_barrier()

  def step(k, carry):
    pltpu.sync_copy(rows.at[pl.ds(w * EW + k * CH, CH)], rv)
    pltpu.sync_copy(rv, acc.at[iv.at[k]], add=True)
    return carry

  lax.fori_loop(0, KCH, step, 0)
  plsc.subcore_barrier()

  @pl.when(s < 10)
  def _():
    pltpu.sync_copy(acc.at[pl.ds(s * NSTRIPE, NSTRIPE)],
                    out.at[c, pl.ds(s * NSTRIPE, NSTRIPE)])


def _sc_gather2(tab, i2, j2):
  mesh = plsc.VectorSubcoreMesh(core_axis_name="c", subcore_axis_name="s")
  f = pl.kernel(
      _sc_gather2_body,
      compiler_params=pltpu.CompilerParams(use_tc_tiling_on_sc=False),
      out_type=[jax.ShapeDtypeStruct((E, NSC), jnp.float32),
                jax.ShapeDtypeStruct((E, NSC), jnp.float32)],
      mesh=mesh,
      scratch_types=[pltpu.VMEM((KCH, CH), jnp.int32),
                     pltpu.VMEM((KCH, CH), jnp.int32),
                     pltpu.VMEM((CH, NSC), jnp.float32)],
  )
  return f(tab, i2, j2)


def _sc_gather1(tab, i2):
  mesh = plsc.VectorSubcoreMesh(core_axis_name="c", subcore_axis_name="s")
  f = pl.kernel(
      _sc_gather1_body,
      compiler_params=pltpu.CompilerParams(use_tc_tiling_on_sc=False),
      out_type=jax.ShapeDtypeStruct((E, NSC), jnp.float32),
      mesh=mesh,
      scratch_types=[pltpu.VMEM((KCH, CH), jnp.int32),
                     pltpu.VMEM((CH, NSC), jnp.float32)],
  )
  return f(tab, i2)


def _sc_scatter(rows, i2, zrows, width):
  mesh = plsc.VectorSubcoreMesh(core_axis_name="c", subcore_axis_name="s")
  f = pl.kernel(
      _sc_scatter_body,
      compiler_params=pltpu.CompilerParams(use_tc_tiling_on_sc=False),
      out_type=jax.ShapeDtypeStruct((2, N, width), jnp.float32),
      mesh=mesh,
      scratch_types=[pltpu.VMEM((KCH, CH), jnp.int32),
                     pltpu.VMEM((CH, width), jnp.float32),
                     pltpu.VMEM_SHARED((N, width), jnp.float32)],
  )
  return f(rows, i2, zrows)


def _tc_species_body(z_ref, emb_ref, out_ref):
  z = z_ref[...]
  ids = lax.broadcasted_iota(jnp.int32, (N, 128), 1)
  oh = jnp.where(ids == z, 1.0, 0.0).astype(jnp.float32)
  out_ref[...] = jnp.dot(oh, emb_ref[...], preferred_element_type=jnp.float32)


def _tc_edge1_body(r_ref, spi_ref, spj_ref, w_ref, lb_ref, rad_ref, sph_ref,
                   c1_ref):
  R = r_ref[...]
  x, y, z = R[:, 0:1], R[:, 1:2], R[:, 2:3]
  r2 = x * x + y * y + z * z
  r = jnp.sqrt(r2 + 1e-12)
  cut = jnp.where(r < CUT, 0.5 * (jnp.cos(jnp.pi * r / CUT) + 1.0), 0.0)
  t = jnp.clip(r / CUT, 1e-06, 1.0 - 1e-06)
  lt = jnp.log(t)
  l1t = jnp.log1p(-t)
  v = lax.broadcasted_iota(jnp.int32, (EBLK, NR), 1).astype(jnp.float32)
  radial = jnp.exp(lb_ref[...] + v * lt + (NR - 1.0 - v) * l1t)
  s3 = math.sqrt(3.0)
  zero = jnp.zeros((EBLK, 7), jnp.float32)
  sph = jnp.concatenate(
      [jnp.ones_like(x), y, z, x, s3 * x * y, s3 * y * z,
       0.5 * (3.0 * z * z - r2), s3 * x * z, 0.5 * s3 * (x * x - y * y),
       zero], axis=1)
  spi = spi_ref[...]
  spj = spj_ref[...]
  X = jnp.concatenate(
      [radial * spi[:, c:c + 1] for c in range(NSP)]
      + [radial * spj[:, c:c + 1] for c in range(NSP)]
      + [radial], axis=1)
  c1 = jnp.dot(X, w_ref[...], preferred_element_type=jnp.float32) * cut
  rad_ref[...] = radial
  sph_ref[...] = sph
  c1_ref[...] = jnp.concatenate(
      [c1, cut, jnp.zeros((EBLK, 15), jnp.float32)], axis=1)


def _tc_nodeA_body(part_ref, scal_ref, ps_ref):
  p = part_ref[0] + part_ref[1]
  ps = 1.0 / (1.0 + p[:, NSC:NSC + 1])
  scal_ref[...] = p[:, 0:NSC] * ps
  ps_ref[...] = jnp.broadcast_to(ps, (N, NSC))


def _tc_edge2_body(rad_ref, sph_ref, c1_ref, sci_ref, spj_ref, w_ref, out_ref):
  radial = rad_ref[...]
  cut = c1_ref[:, NSC:NSC + 1]
  sci = sci_ref[...]
  spj = spj_ref[...]
  X = jnp.concatenate(
      [radial * sci[:, c:c + 1] for c in range(NSC)]
      + [radial * spj[:, c:c + 1] for c in range(NSP)]
      + [radial], axis=1)
  c2 = jnp.dot(X, w_ref[...], preferred_element_type=jnp.float32) * cut
  sph = sph_ref[...]
  l_of = (0, 1, 1, 1, 2, 2, 2, 2, 2)
  pieces = [c2[:, 8 * l_of[lm]:8 * l_of[lm] + 8] * sph[:, lm:lm + 1]
            for lm in range(NLM)]
  pieces.append(jnp.zeros((EBLK, 8), jnp.float32))
  out_ref[...] = jnp.concatenate(pieces, axis=1)


def _tc_nodeB_body(part_ref, scal_ref, ps_ref, wtr_ref, btr_ref, s_ref, b_ref,
                   out_ref):
  ps = ps_ref[...]
  sphf = (part_ref[0] + part_ref[1]) * ps[:, 0:1]
  sq = sphf * sphf
  l0 = sq[:, 0:8]
  l1 = sq[:, 8:16] + sq[:, 16:24] + sq[:, 24:32]
  l2 = (sq[:, 32:40] + sq[:, 40:48] + sq[:, 48:56] + sq[:, 56:64]
        + sq[:, 64:72])
  trace = jnp.concatenate([l0, l1, l2], axis=1)
  s2 = scal_ref[...] + jnp.dot(trace, wtr_ref[...],
                               preferred_element_type=jnp.float32) + btr_ref[...]
  mu = jnp.mean(s2, axis=1, keepdims=True)
  d = s2 - mu
  var = jnp.mean(d * d, axis=1, keepdims=True)
  out_ref[...] = d / jnp.sqrt(var + 1e-06) * s_ref[...] + b_ref[...]


def _tc_edge3_body(rad_ref, c1_ref, sli_ref, slj_ref, w_ref, out_ref):
  radial = rad_ref[...]
  cut = c1_ref[:, NSC:NSC + 1]
  sli = sli_ref[...]
  slj = slj_ref[...]
  X = jnp.concatenate(
      [radial * sli[:, c:c + 1] for c in range(NSC)]
      + [radial * slj[:, c:c + 1] for c in range(NSC)]
      + [radial], axis=1)
  out_ref[...] = jnp.dot(X, w_ref[...],
                         preferred_element_type=jnp.float32) * cut


def _tc_nodeC_body(part_ref, scal_ref, ps_ref, s_ref, b_ref, out_ref):
  s2 = scal_ref[...] + (part_ref[0] + part_ref[1]) * ps_ref[...]
  mu = jnp.mean(s2, axis=1, keepdims=True)
  d = s2 - mu
  var = jnp.mean(d * d, axis=1, keepdims=True)
  out_ref[...] = d / jnp.sqrt(var + 1e-06) * s_ref[...] + b_ref[...]


def _full(shape):
  return pl.BlockSpec(shape, lambda: tuple(0 for _ in shape))


def _eblk(width):
  return pl.BlockSpec((EBLK, width), lambda b: (b, 0))


def _const(shape):
  return pl.BlockSpec(shape, lambda b: tuple(0 for _ in shape))


def kernel(R_ij, i, j, Z_i, node_to_graph, pair_mask, node_mask, embed_table,
           W1, b1, W2, b2, W_tr, b_tr, ln1_s, ln1_b, Wq1, bq1, Wq2, bq2,
           W3, b3, ln2_s, ln2_b):
  f32 = jnp.float32
  i2 = i.reshape(NWORK, KCH, CH)
  j2 = j.reshape(NWORK, KCH, CH)
  z2 = Z_i.reshape(N, 1)
  embP = jnp.zeros((128, NSC), f32).at[:100, :NSP].set(embed_table)
  w1big = jnp.concatenate([W1.reshape(2 * NSP * NR, NSC),
                           b1.reshape(NR, NSC)], axis=0)
  w2big = jnp.concatenate([W2.reshape((NSC + NSP) * NR, 24),
                           b2.reshape(NR, 24)], axis=0)
  w3big = jnp.concatenate([W3.reshape(2 * NSC * NR, NSC),
                           b3.reshape(NR, NSC)], axis=0)
  lb = jnp.asarray(_LOG_BINOM, f32).reshape(1, NR)
  btr = b_tr.reshape(1, NSC)
  ln1s = ln1_s.reshape(1, NSC)
  ln1b = ln1_b.reshape(1, NSC)
  ln2s = ln2_s.reshape(1, NSC)
  ln2b = ln2_b.reshape(1, NSC)
  z32 = jnp.zeros((NSTRIPE, 32), f32)
  z80 = jnp.zeros((NSTRIPE, 80), f32)
  z16 = jnp.zeros((NSTRIPE, NSC), f32)

  sp16 = pl.pallas_call(
      _tc_species_body,
      out_shape=jax.ShapeDtypeStruct((N, NSC), f32),
  )(z2, embP)

  spi, spj = _sc_gather2(sp16, i2, j2)

  radial, sph, c1ext = pl.pallas_call(
      _tc_edge1_body,
      grid=(EGRID,),
      in_specs=[_eblk(3), _eblk(NSC), _eblk(NSC), _const((2 * NSP * NR + NR,
                                                          NSC)),
                _const((1, NR))],
      out_specs=[_eblk(NR), _eblk(NSC), _eblk(32)],
      out_shape=[jax.ShapeDtypeStruct((E, NR), f32),
                 jax.ShapeDtypeStruct((E, NSC), f32),
                 jax.ShapeDtypeStruct((E, 32), f32)],
  )(R_ij, spi, spj, w1big, lb)

  part1 = _sc_scatter(c1ext, i2, z32, 32)

  scalar, ps = pl.pallas_call(
      _tc_nodeA_body,
      out_shape=[jax.ShapeDtypeStruct((N, NSC), f32),
                 jax.ShapeDtypeStruct((N, NSC), f32)],
  )(part1)

  sci = _sc_gather1(scalar, i2)

  c2sph = pl.pallas_call(
      _tc_edge2_body,
      grid=(EGRID,),
      in_specs=[_eblk(NR), _eblk(NSC), _eblk(32), _eblk(NSC), _eblk(NSC),
                _const(((NSC + NSP) * NR + NR, 24))],
      out_specs=_eblk(80),
      out_shape=jax.ShapeDtypeStruct((E, 80), f32),
  )(radial, sph, c1ext, sci, spj, w2big)

  part2 = _sc_scatter(c2sph, i2, z80, 80)

  scal_ln = pl.pallas_call(
      _tc_nodeB_body,
      out_shape=jax.ShapeDtypeStruct((N, NSC), f32),
  )(part2, scalar, ps, W_tr, btr, ln1s, ln1b)

  sli, slj = _sc_gather2(scal_ln, i2, j2)

  radp = pl.pallas_call(
      _tc_edge3_body,
      grid=(EGRID,),
      in_specs=[_eblk(NR), _eblk(32), _eblk(NSC), _eblk(NSC),
                _const((2 * NSC * NR + NR, NSC))],
      out_specs=_eblk(NSC),
      out_shape=jax.ShapeDtypeStruct((E, NSC), f32),
  )(radial, c1ext, sli, slj, w3big)

  part3 = _sc_scatter(radp, i2, z16, NSC)

  out = pl.pallas_call(
      _tc_nodeC_body,
      out_shape=jax.ShapeDtypeStruct((N, NSC), f32),
  )(part3, scal_ln, ps, ln2s, ln2b)
  return out
